# baseline (speedup 1.0000x reference)
import jax
import jax.numpy as jnp
from jax.experimental import pallas as pl
from jax.experimental.pallas import tpu as pltpu

_S, _P, _C = 64, 2048, 32
_A, _D, _E = 2048, 128, 64
_RPS = _A // _S

_HI = jax.lax.Precision.DEFAULT

_CB = 16
_T = _A // _CB + _S


_BR = 256


def _iota(shape, dim):
    return jax.lax.broadcasted_iota(jnp.int32, shape, dim)


def _precompute_kernel(scene_ref, dyn_ref, sid_ref, wsn_ref, bsn_ref, wdf_ref,
                       bdf_ref, z1_ref, z2_ref, sfor_ref, row0_ref, hi_ref,
                       perm_ref):
    z1_ref[0] = (jax.lax.dot_general(
        wsn_ref[...], scene_ref[0], (((0,), (1,)), ((), ())),
        preferred_element_type=jnp.float32, precision=_HI)
        + bsn_ref[...]).astype(jnp.bfloat16)
    z2_ref[0] = jnp.dot(dyn_ref[0], wdf_ref[...],
                        preferred_element_type=jnp.float32,
                        precision=_HI) + bdf_ref[...]

    @pl.when(pl.program_id(0) == 0)
    def _():
        sid = sid_ref[...]
        arow = _iota((1, _A), 1)
        srow = _iota((1, _S), 1)
        scol = _iota((_S, 1), 0)
        occT = (sid == scol).astype(jnp.int32)
        count_col = jnp.sum(occT, axis=1, keepdims=True)
        count_row = jnp.transpose(count_col)
        ltT = (srow < scol).astype(jnp.int32)
        seg_start_col = jnp.sum(ltT * count_row, axis=1, keepdims=True)
        seg_end_col = seg_start_col + count_col
        seg_start_row = jnp.transpose(seg_start_col)
        seg_end_row = jnp.transpose(seg_end_col)

        pos_blocks = []
        for b in range(_A // _BR):
            sid_blk_col = jnp.transpose(sid[:, b * _BR:(b + 1) * _BR])
            idx_col = _iota((_BR, 1), 0) + b * _BR
            same = sid == sid_blk_col
            earlier = arow < idx_col
            rank_col = jnp.sum((same & earlier).astype(jnp.int32),
                               axis=1, keepdims=True)
            onehot_blk = (sid_blk_col == srow).astype(jnp.int32)
            st = jnp.sum(onehot_blk * seg_start_row, axis=1, keepdims=True)
            pos_blocks.append(st + rank_col)
        pos_row = jnp.transpose(jnp.concatenate(pos_blocks, axis=0))
        for b in range(_A // _BR):
            r_col = _iota((_BR, 1), 0) + b * _BR
            eq = pos_row == r_col
            perm_ref[b * _BR:(b + 1) * _BR, :] = jnp.sum(
                jnp.where(eq, arow, 0), axis=1, keepdims=True)

        nchunk_col = (count_col + _CB - 1) // _CB
        nchunk_row = jnp.transpose(nchunk_col)
        base_col = jnp.sum(ltT * nchunk_row, axis=1, keepdims=True)
        base_row = jnp.transpose(base_col)
        total = jnp.sum(nchunk_col)
        t_col = _iota((_T, 1), 0)
        elig = (t_col >= base_row) & (nchunk_row > 0)
        sfor_col = jnp.maximum(
            jnp.max(jnp.where(elig, srow, -1), axis=1, keepdims=True), 0)
        oh = (sfor_col == srow).astype(jnp.int32)
        kk = t_col - jnp.sum(oh * base_row, axis=1, keepdims=True)
        sfor_ref[...] = sfor_col
        row0_ref[...] = (jnp.sum(oh * seg_start_row, axis=1, keepdims=True)
                         + kk * _CB)
        hi_ref[...] = jnp.where(
            t_col < total,
            jnp.sum(oh * seg_end_row, axis=1, keepdims=True), 0)


def _attend_kernel(cs_ref, row0_ref, hi_ref, perm_ref, z1_ref,
                   scene_ref, z2_ref, wrow_ref, out_ref, a2_scr):
    t = pl.program_id(0)
    row0 = row0_ref[t]
    hi = hi_ref[t]

    @pl.when(hi > 0)
    def _():
        for i in range(_CB):
            rc = jnp.minimum(row0 + i, _A - 1)
            a2_scr[i, :] = z2_ref[perm_ref[rc], :]
        a2t = jnp.transpose(a2_scr[...].astype(jnp.bfloat16))
        wrow = wrow_ref[...]
        z1 = z1_ref[0]
        zero = jnp.zeros((), jnp.bfloat16)
        rows = []
        for i in range(_CB):
            yi = jnp.maximum(z1 + a2t[:, i:i + 1], zero)
            rows.append(jnp.dot(wrow, yi,
                                preferred_element_type=jnp.float32,
                                precision=_HI))
        logits = jnp.concatenate(rows, axis=0)
        m = jnp.max(logits, axis=1, keepdims=True)
        e = jnp.exp(logits - m)
        s = jnp.sum(e, axis=1, keepdims=True)
        pooled = jnp.dot(e, scene_ref[0],
                         preferred_element_type=jnp.float32,
                         precision=_HI)
        res = pooled / s
        for i in range(_CB):
            r = row0 + i

            @pl.when(r < hi)
            def _():
                rc = jnp.minimum(r, _A - 1)
                out_ref[pl.ds(perm_ref[rc], 1), :] = res[i:i + 1, :]


def kernel(global_scene, scene_idx, dynamic_encoding, W_sn, b_sn, W_df, b_df,
           w_fc, b_fc):
    del b_fc
    scene_idx = scene_idx.astype(jnp.int32)

    z1, z2, sfor, row0_arr, hi_arr, perm = pl.pallas_call(
        _precompute_kernel,
        grid=(_S,),
        in_specs=[
            pl.BlockSpec((1, _P, _C), lambda s: (s, 0, 0)),
            pl.BlockSpec((1, _RPS, _D), lambda s: (s, 0, 0)),
            pl.BlockSpec((1, _A), lambda s: (0, 0)),
            pl.BlockSpec((_C, _E), lambda s: (0, 0)),
            pl.BlockSpec((_E, 1), lambda s: (0, 0)),
            pl.BlockSpec((_D, _E), lambda s: (0, 0)),
            pl.BlockSpec((1, _E), lambda s: (0, 0)),
        ],
        out_specs=[
            pl.BlockSpec((1, _E, _P), lambda s: (s, 0, 0)),
            pl.BlockSpec((1, _RPS, _E), lambda s: (s, 0, 0)),
            pl.BlockSpec((_T, 1), lambda s: (0, 0)),
            pl.BlockSpec((_T, 1), lambda s: (0, 0)),
            pl.BlockSpec((_T, 1), lambda s: (0, 0)),
            pl.BlockSpec((_A, 1), lambda s: (0, 0)),
        ],
        out_shape=[
            jax.ShapeDtypeStruct((_S, _E, _P), jnp.bfloat16),
            jax.ShapeDtypeStruct((_S, _RPS, _E), jnp.float32),
            jax.ShapeDtypeStruct((_T, 1), jnp.int32),
            jax.ShapeDtypeStruct((_T, 1), jnp.int32),
            jax.ShapeDtypeStruct((_T, 1), jnp.int32),
            jax.ShapeDtypeStruct((_A, 1), jnp.int32),
        ],
        compiler_params=pltpu.CompilerParams(
            dimension_semantics=("arbitrary",)),
        name="cma_precompute",
    )(global_scene, dynamic_encoding.reshape(_S, _RPS, _D),
      scene_idx.reshape(1, _A), W_sn, b_sn.reshape(_E, 1), W_df,
      b_df.reshape(1, _E))

    z2 = z2.reshape(_A, _E)
    sfor = sfor.reshape(_T)
    row0_arr = row0_arr.reshape(_T)
    hi_arr = hi_arr.reshape(_T)
    perm = perm.reshape(_A)

    out = pl.pallas_call(
        _attend_kernel,
        grid_spec=pltpu.PrefetchScalarGridSpec(
            num_scalar_prefetch=4,
            grid=(_T,),
            in_specs=[
                pl.BlockSpec((1, _E, _P), lambda t, cs, r0, hi, prm: (cs[t], 0, 0)),
                pl.BlockSpec((1, _P, _C), lambda t, cs, r0, hi, prm: (cs[t], 0, 0)),
                pl.BlockSpec((_A, _E), lambda t, cs, r0, hi, prm: (0, 0)),
                pl.BlockSpec((1, _E), lambda t, cs, r0, hi, prm: (0, 0)),
            ],
            out_specs=pl.BlockSpec((_A, _C), lambda t, cs, r0, hi, prm: (0, 0)),
            scratch_shapes=[pltpu.VMEM((_CB, _E), jnp.float32)],
        ),
        out_shape=jax.ShapeDtypeStruct((_A, _C), jnp.float32),
        compiler_params=pltpu.CompilerParams(
            dimension_semantics=("arbitrary",)),
        name="cma_attend",
    )(sfor, row0_arr, hi_arr, perm, z1, global_scene, z2,
      w_fc.reshape(1, _E).astype(jnp.bfloat16))

    return out
